# Initial kernel scaffold; baseline (speedup 1.0000x reference)
import jax
import jax.numpy as jnp
from jax.experimental import pallas as pl


def kernel(x, adjs, W1, W2, W3):
    raise NotImplementedError("write your pallas kernel here")


# speedup vs baseline: 2.6363x; 2.6363x over previous
import jax
import jax.numpy as jnp
from jax.experimental import pallas as pl

_S = 1024
_F = 512


def _fused_sage_body(x_ref, adj_ref, w1_ref, w2_ref, w3_ref, out_ref):
    adj = adj_ref[0]
    h = x_ref[...]
    for i, w_ref in enumerate((w1_ref, w2_ref, w3_ref)):
        ax = jnp.dot(adj, h, preferred_element_type=jnp.float32)
        h = jax.lax.dot_general(
            h + ax, w_ref[...],
            (((1,), (1,)), ((), ())),
            preferred_element_type=jnp.float32)
        if i < 2:
            h = jnp.maximum(h, 0.0)
    m = jnp.max(h, axis=1, keepdims=True)
    lse = jnp.log(jnp.sum(jnp.exp(h - m), axis=1, keepdims=True)) + m
    out_ref[...] = h - lse


def kernel(x, adjs, W1, W2, W3):
    nblocks = adjs.shape[0]
    return pl.pallas_call(
        _fused_sage_body,
        grid=(nblocks,),
        in_specs=[
            pl.BlockSpec((_S, _F), lambda i: (i, 0)),
            pl.BlockSpec((1, _S, _S), lambda i: (i, 0, 0)),
            pl.BlockSpec((_F, _F), lambda i: (0, 0)),
            pl.BlockSpec((_F, _F), lambda i: (0, 0)),
            pl.BlockSpec((_F, _F), lambda i: (0, 0)),
        ],
        out_specs=pl.BlockSpec((_S, _F), lambda i: (i, 0)),
        out_shape=jax.ShapeDtypeStruct(x.shape, x.dtype),
    )(x, adjs, W1, W2, W3)


# speedup vs baseline: 2.6402x; 1.0015x over previous
import jax
import jax.numpy as jnp
from jax.experimental import pallas as pl

_S = 1024
_F = 512


def _fused_sage_body(x_ref, adj_ref, w1_ref, w2_ref, w3_ref, out_ref):
    adj = adj_ref[0].astype(jnp.bfloat16)
    h = x_ref[...]
    for i, w_ref in enumerate((w1_ref, w2_ref, w3_ref)):
        ax = jnp.dot(adj, h.astype(jnp.bfloat16),
                     preferred_element_type=jnp.float32)
        h = jax.lax.dot_general(
            h + ax, w_ref[...],
            (((1,), (1,)), ((), ())),
            preferred_element_type=jnp.float32)
        if i < 2:
            h = jnp.maximum(h, 0.0)
    m = jnp.max(h, axis=1, keepdims=True)
    lse = jnp.log(jnp.sum(jnp.exp(h - m), axis=1, keepdims=True)) + m
    out_ref[...] = h - lse


def kernel(x, adjs, W1, W2, W3):
    nblocks = adjs.shape[0]
    return pl.pallas_call(
        _fused_sage_body,
        grid=(nblocks,),
        in_specs=[
            pl.BlockSpec((_S, _F), lambda i: (i, 0)),
            pl.BlockSpec((1, _S, _S), lambda i: (i, 0, 0)),
            pl.BlockSpec((_F, _F), lambda i: (0, 0)),
            pl.BlockSpec((_F, _F), lambda i: (0, 0)),
            pl.BlockSpec((_F, _F), lambda i: (0, 0)),
        ],
        out_specs=pl.BlockSpec((_S, _F), lambda i: (i, 0)),
        out_shape=jax.ShapeDtypeStruct(x.shape, x.dtype),
    )(x, adjs, W1, W2, W3)
